# baseline (device time: 113166 ns/iter reference)
import jax
import jax.numpy as jnp
from jax import lax
from jax.experimental import pallas as pl
from jax.experimental.pallas import tpu as pltpu

N_DEV = 4
M_PER = 1024
HALF = M_PER // 2
QTR = M_PER // 4
FWD = 128
N_FWD = HALF // FWD
K = 4096
N_PER = 2048
KB = 256
N_KB = K // KB
XB = 256
N_XB = M_PER // XB


def kernel(x, w_mat, scale_x, scale_w):
    s = (scale_x * scale_w).reshape(1, 1)

    def body(x_hbm, w_hbm, s_ref, out_init, out_hbm,
             x8, xtmp, w8, wtmp, recv_l, recv_r, far, ostage,
             xcopy_sems, wcopy_sems, ocopy_sems, send_sems, recv_sems):
        my_pos = lax.axis_index("i")
        left = lax.rem(my_pos + N_DEV - 1, N_DEV)
        right = lax.rem(my_pos + 1, N_DEV)
        far_origin = lax.rem(my_pos + 2, N_DEV)

        def rdma(src, dst, i, dev):
            return pltpu.make_async_remote_copy(
                src_ref=src, dst_ref=dst,
                send_sem=send_sems.at[i], recv_sem=recv_sems.at[i],
                device_id=(dev,), device_id_type=pl.DeviceIdType.MESH,
            )

        s_ra1 = rdma(x8.at[pl.ds(0, QTR)], recv_l.at[pl.ds(0, QTR)],
                     0, right)
        s_ra2 = rdma(x8.at[pl.ds(QTR, QTR)], recv_l.at[pl.ds(QTR, QTR)],
                     1, right)
        s_rb = rdma(x8.at[pl.ds(HALF, HALF)], recv_l.at[pl.ds(HALF, HALF)],
                    2, right)
        s_lb1 = rdma(x8.at[pl.ds(HALF, QTR)], recv_r.at[pl.ds(HALF, QTR)],
                     3, left)
        s_lb2 = rdma(x8.at[pl.ds(HALF + QTR, QTR)],
                     recv_r.at[pl.ds(HALF + QTR, QTR)], 4, left)
        s_la = rdma(x8.at[pl.ds(0, HALF)], recv_r.at[pl.ds(0, HALF)],
                    5, left)
        f_r = [rdma(recv_l.at[pl.ds(k * FWD, FWD)],
                    far.at[pl.ds(k * FWD, FWD)], 6 + k, right)
               for k in range(N_FWD)]
        f_l = [rdma(recv_r.at[pl.ds(HALF + k * FWD, FWD)],
                    far.at[pl.ds(HALF + k * FWD, FWD)], 10 + k, left)
               for k in range(N_FWD)]

        xorder = (0, 2, 1, 3)

        def x_copy(xb, slot):
            return pltpu.make_async_copy(
                x_hbm.at[pl.ds(xb * XB, XB), :],
                xtmp.at[slot],
                xcopy_sems.at[slot],
            )

        x_copy(xorder[0], 0).start()
        x_copy(xorder[1], 1).start()
        for i, xb in enumerate(xorder):
            x_copy(xb, i % 2).wait()
            x8[pl.ds(xb * XB, XB), :] = xtmp[i % 2].astype(
                jnp.float8_e4m3fn
            )
            if i + 2 < N_XB:
                x_copy(xorder[i + 2], i % 2).start()
            if i == 0:
                barrier_sem = pltpu.get_barrier_semaphore()
                for nbr in (left, right):
                    pl.semaphore_signal(
                        barrier_sem, inc=1,
                        device_id=(nbr,),
                        device_id_type=pl.DeviceIdType.MESH,
                    )
                pl.semaphore_wait(barrier_sem, 2)
                s_ra1.start()
            elif i == 1:
                s_lb1.start()
            elif i == 2:
                s_ra2.start()
        s_lb2.start()
        s_rb.start()
        s_la.start()

        col0 = my_pos * N_PER

        def w_copy(kb, slot):
            return pltpu.make_async_copy(
                w_hbm.at[pl.ds(kb * KB, KB), pl.ds(col0, N_PER)],
                wtmp.at[slot],
                wcopy_sems.at[slot],
            )

        w_copy(0, 0).start()
        w_copy(1, 1).start()
        for kb in range(N_KB):
            w_copy(kb, kb % 2).wait()
            w8[pl.ds(kb * KB, KB), :] = wtmp[kb % 2].astype(
                jnp.float8_e4m3fn
            )
            if kb + 2 < N_KB:
                w_copy(kb + 2, kb % 2).start()

        scale = s_ref[0, 0]
        out_copies = {}

        def emit(src, r0, nr, origin, slot, osem):
            acc = jnp.dot(src[pl.ds(r0, nr), :], w8[...],
                          preferred_element_type=jnp.float32)
            ostage[slot, pl.ds(r0, nr), :] = jnp.maximum(acc * scale, 0.0)
            cp = pltpu.make_async_copy(
                ostage.at[slot, pl.ds(r0, nr)],
                out_hbm.at[pl.ds(origin * M_PER + r0, nr)],
                ocopy_sems.at[osem],
            )
            cp.start()
            out_copies[osem] = cp

        def owait(osem):
            out_copies[osem].wait()

        emit(x8, 0, M_PER, my_pos, 0, 0)

        s_ra1.wait_recv()
        s_ra2.wait_recv()
        for op in f_r:
            op.start()
        s_lb1.wait_recv()
        s_lb2.wait_recv()
        for op in f_l:
            op.start()
        emit(recv_l, 0, HALF, left, 1, 1)
        emit(recv_r, HALF, HALF, right, 1, 2)
        s_rb.wait_recv()
        owait(0)
        emit(recv_l, HALF, HALF, left, 0, 3)
        s_la.wait_recv()
        emit(recv_r, 0, HALF, right, 0, 4)

        f_r[0].wait_recv()
        owait(1)
        emit(far, 0, FWD, far_origin, 1, 5)
        f_l[0].wait_recv()
        owait(2)
        emit(far, HALF, FWD, far_origin, 1, 6)
        for k in range(1, N_FWD):
            f_r[k].wait_recv()
            emit(far, k * FWD, FWD, far_origin, 1, 5 + 2 * k)
            f_l[k].wait_recv()
            emit(far, HALF + k * FWD, FWD, far_origin, 1, 6 + 2 * k)

        for osem in range(3, 13):
            owait(osem)
        for op in [s_ra1, s_ra2, s_rb, s_lb1, s_lb2, s_la] + f_r + f_l:
            op.wait_send()

    return pl.pallas_call(
        body,
        out_shape=jax.ShapeDtypeStruct((N_DEV * M_PER, N_PER), jnp.float32),
        in_specs=[
            pl.BlockSpec(memory_space=pltpu.HBM),
            pl.BlockSpec(memory_space=pltpu.HBM),
            pl.BlockSpec(memory_space=pltpu.SMEM),
            pl.BlockSpec(memory_space=pltpu.HBM),
        ],
        out_specs=pl.BlockSpec(memory_space=pltpu.HBM),
        input_output_aliases={3: 0},
        scratch_shapes=[
            pltpu.VMEM((M_PER, K), jnp.float8_e4m3fn),
            pltpu.VMEM((2, XB, K), jnp.float32),
            pltpu.VMEM((K, N_PER), jnp.float8_e4m3fn),
            pltpu.VMEM((2, KB, N_PER), jnp.float32),
            pltpu.VMEM((M_PER, K), jnp.float8_e4m3fn),
            pltpu.VMEM((M_PER, K), jnp.float8_e4m3fn),
            pltpu.VMEM((M_PER, K), jnp.float8_e4m3fn),
            pltpu.VMEM((2, M_PER, N_PER), jnp.float32),
            pltpu.SemaphoreType.DMA((2,)),
            pltpu.SemaphoreType.DMA((2,)),
            pltpu.SemaphoreType.DMA((13,)),
            pltpu.SemaphoreType.DMA((14,)),
            pltpu.SemaphoreType.DMA((14,)),
        ],
        compiler_params=pltpu.CompilerParams(
            collective_id=0,
            vmem_limit_bytes=60 * 1024 * 1024,
        ),
    )(x, w_mat, s, jnp.zeros((N_DEV * M_PER, N_PER), jnp.float32))


# device time: 101618 ns/iter; 1.1136x vs baseline; 1.1136x over previous
import jax
import jax.numpy as jnp
from jax import lax
from jax.experimental import pallas as pl
from jax.experimental.pallas import tpu as pltpu

N_DEV = 4
M_PER = 1024
HALF = M_PER // 2
QTR = M_PER // 4
FWD = 128
N_FWD = HALF // FWD
K = 4096
N_PER = 2048
KB = 256
N_KB = K // KB
XB = 256
N_XB = M_PER // XB


def kernel(x, w_mat, scale_x, scale_w):
    s = (scale_x * scale_w).reshape(1, 1)

    def body(x_hbm, w_hbm, s_ref, out_hbm,
             x8, xtmp, w8, wtmp, recv_l, recv_r, far, ostage,
             xcopy_sems, wcopy_sems, ocopy_sems, send_sems, recv_sems):
        my_pos = lax.axis_index("i")
        left = lax.rem(my_pos + N_DEV - 1, N_DEV)
        right = lax.rem(my_pos + 1, N_DEV)
        far_origin = lax.rem(my_pos + 2, N_DEV)

        def rdma(src, dst, i, dev):
            return pltpu.make_async_remote_copy(
                src_ref=src, dst_ref=dst,
                send_sem=send_sems.at[i], recv_sem=recv_sems.at[i],
                device_id=(dev,), device_id_type=pl.DeviceIdType.MESH,
            )

        s_ra1 = rdma(x8.at[pl.ds(0, QTR)], recv_l.at[pl.ds(0, QTR)],
                     0, right)
        s_ra2 = rdma(x8.at[pl.ds(QTR, QTR)], recv_l.at[pl.ds(QTR, QTR)],
                     1, right)
        s_rb = rdma(x8.at[pl.ds(HALF, HALF)], recv_l.at[pl.ds(HALF, HALF)],
                    2, right)
        s_lb1 = rdma(x8.at[pl.ds(HALF, QTR)], recv_r.at[pl.ds(HALF, QTR)],
                     3, left)
        s_lb2 = rdma(x8.at[pl.ds(HALF + QTR, QTR)],
                     recv_r.at[pl.ds(HALF + QTR, QTR)], 4, left)
        s_la = rdma(x8.at[pl.ds(0, HALF)], recv_r.at[pl.ds(0, HALF)],
                    5, left)
        f_r = [rdma(recv_l.at[pl.ds(k * FWD, FWD)],
                    far.at[pl.ds(k * FWD, FWD)], 6 + k, right)
               for k in range(N_FWD)]
        f_l = [rdma(recv_r.at[pl.ds(HALF + k * FWD, FWD)],
                    far.at[pl.ds(HALF + k * FWD, FWD)], 10 + k, left)
               for k in range(N_FWD)]

        xorder = (0, 2, 1, 3)

        def x_copy(xb, slot):
            return pltpu.make_async_copy(
                x_hbm.at[pl.ds(xb * XB, XB), :],
                xtmp.at[slot],
                xcopy_sems.at[slot],
            )

        x_copy(xorder[0], 0).start()
        x_copy(xorder[1], 1).start()
        for i, xb in enumerate(xorder):
            x_copy(xb, i % 2).wait()
            x8[pl.ds(xb * XB, XB), :] = xtmp[i % 2].astype(
                jnp.float8_e4m3fn
            )
            if i + 2 < N_XB:
                x_copy(xorder[i + 2], i % 2).start()
            if i == 0:
                barrier_sem = pltpu.get_barrier_semaphore()
                for nbr in (left, right):
                    pl.semaphore_signal(
                        barrier_sem, inc=1,
                        device_id=(nbr,),
                        device_id_type=pl.DeviceIdType.MESH,
                    )
                pl.semaphore_wait(barrier_sem, 2)
                s_ra1.start()
            elif i == 1:
                s_lb1.start()
            elif i == 2:
                s_ra2.start()
        s_lb2.start()
        s_rb.start()
        s_la.start()

        col0 = my_pos * N_PER

        def w_copy(kb, slot):
            return pltpu.make_async_copy(
                w_hbm.at[pl.ds(kb * KB, KB), pl.ds(col0, N_PER)],
                wtmp.at[slot],
                wcopy_sems.at[slot],
            )

        w_copy(0, 0).start()
        w_copy(1, 1).start()
        for kb in range(N_KB):
            w_copy(kb, kb % 2).wait()
            w8[pl.ds(kb * KB, KB), :] = wtmp[kb % 2].astype(
                jnp.float8_e4m3fn
            )
            if kb + 2 < N_KB:
                w_copy(kb + 2, kb % 2).start()

        scale = s_ref[0, 0]
        out_copies = {}

        def emit(src, r0, nr, origin, slot, osem):
            acc = jnp.dot(src[pl.ds(r0, nr), :], w8[...],
                          preferred_element_type=jnp.float32)
            ostage[slot, pl.ds(r0, nr), :] = jnp.maximum(acc * scale, 0.0)
            cp = pltpu.make_async_copy(
                ostage.at[slot, pl.ds(r0, nr)],
                out_hbm.at[pl.ds(origin * M_PER + r0, nr)],
                ocopy_sems.at[osem],
            )
            cp.start()
            out_copies[osem] = cp

        def owait(osem):
            out_copies[osem].wait()

        emit(x8, 0, M_PER, my_pos, 0, 0)

        s_ra1.wait_recv()
        s_ra2.wait_recv()
        for op in f_r:
            op.start()
        s_lb1.wait_recv()
        s_lb2.wait_recv()
        for op in f_l:
            op.start()
        emit(recv_l, 0, HALF, left, 1, 1)
        emit(recv_r, HALF, HALF, right, 1, 2)
        s_rb.wait_recv()
        owait(0)
        emit(recv_l, HALF, HALF, left, 0, 3)
        s_la.wait_recv()
        emit(recv_r, 0, HALF, right, 0, 4)

        f_r[0].wait_recv()
        owait(1)
        emit(far, 0, FWD, far_origin, 1, 5)
        f_l[0].wait_recv()
        owait(2)
        emit(far, HALF, FWD, far_origin, 1, 6)
        for k in range(1, N_FWD):
            f_r[k].wait_recv()
            emit(far, k * FWD, FWD, far_origin, 1, 5 + 2 * k)
            f_l[k].wait_recv()
            emit(far, HALF + k * FWD, FWD, far_origin, 1, 6 + 2 * k)

        for osem in range(3, 13):
            owait(osem)
        for op in [s_ra1, s_ra2, s_rb, s_lb1, s_lb2, s_la] + f_r + f_l:
            op.wait_send()

    return pl.pallas_call(
        body,
        out_shape=jax.ShapeDtypeStruct((N_DEV * M_PER, N_PER), jnp.float32),
        in_specs=[
            pl.BlockSpec(memory_space=pltpu.HBM),
            pl.BlockSpec(memory_space=pltpu.HBM),
            pl.BlockSpec(memory_space=pltpu.SMEM),
        ],
        out_specs=pl.BlockSpec(memory_space=pltpu.HBM),
        scratch_shapes=[
            pltpu.VMEM((M_PER, K), jnp.float8_e4m3fn),
            pltpu.VMEM((2, XB, K), jnp.float32),
            pltpu.VMEM((K, N_PER), jnp.float8_e4m3fn),
            pltpu.VMEM((2, KB, N_PER), jnp.float32),
            pltpu.VMEM((M_PER, K), jnp.float8_e4m3fn),
            pltpu.VMEM((M_PER, K), jnp.float8_e4m3fn),
            pltpu.VMEM((M_PER, K), jnp.float8_e4m3fn),
            pltpu.VMEM((2, M_PER, N_PER), jnp.float32),
            pltpu.SemaphoreType.DMA((2,)),
            pltpu.SemaphoreType.DMA((2,)),
            pltpu.SemaphoreType.DMA((13,)),
            pltpu.SemaphoreType.DMA((14,)),
            pltpu.SemaphoreType.DMA((14,)),
        ],
        compiler_params=pltpu.CompilerParams(
            collective_id=0,
            vmem_limit_bytes=60 * 1024 * 1024,
        ),
    )(x, w_mat, s)
